# baseline (speedup 1.0000x reference)
import jax
import jax.numpy as jnp
from jax.experimental import pallas as pl
from jax.experimental.pallas import tpu as pltpu

_N = 10000
_F = 128
_H = 32
_C = 8
_TM = 200


def _fused_kernel(adj_ref, x_ref, w1_ref, w2_ref, lsm_ref, z_ref,
                  s1_ref, h2_ref):
    p = pl.program_id(0)
    i = pl.program_id(1)

    @pl.when((p == 0) & (i == 0))
    def _():
        s1_ref[...] = jnp.dot(x_ref[...], w1_ref[...],
                              preferred_element_type=jnp.float32)

    @pl.when(p == 0)
    def _():
        h = jnp.maximum(
            jnp.dot(adj_ref[...], s1_ref[...],
                    preferred_element_type=jnp.float32), 0.0)
        h2_ref[pl.ds(i * _TM, _TM), :] = jnp.dot(
            h, w2_ref[...], preferred_element_type=jnp.float32)

    @pl.when(p == 1)
    def _():
        z = jnp.dot(adj_ref[...], h2_ref[...],
                    preferred_element_type=jnp.float32)
        z_ref[...] = z
        m = jnp.max(z, axis=1, keepdims=True)
        lse = jnp.log(jnp.sum(jnp.exp(z - m), axis=1, keepdims=True)) + m
        lsm_ref[...] = z - lse


def kernel(x, adj, W1, W2):
    w1 = W1.reshape(_F, _H)
    w2 = W2.reshape(_H, _C)

    lsm, z = pl.pallas_call(
        _fused_kernel,
        grid=(2, _N // _TM),
        in_specs=[
            pl.BlockSpec((_TM, _N), lambda p, i: (i, 0)),
            pl.BlockSpec((_N, _F), lambda p, i: (0, 0)),
            pl.BlockSpec((_F, _H), lambda p, i: (0, 0)),
            pl.BlockSpec((_H, _C), lambda p, i: (0, 0)),
        ],
        out_specs=[
            pl.BlockSpec((_TM, _C), lambda p, i: (p * i, 0)),
            pl.BlockSpec((_TM, _C), lambda p, i: (p * i, 0)),
        ],
        out_shape=[
            jax.ShapeDtypeStruct((_N, _C), jnp.float32),
            jax.ShapeDtypeStruct((_N, _C), jnp.float32),
        ],
        scratch_shapes=[
            pltpu.VMEM((_N, _H), jnp.float32),
            pltpu.VMEM((_N, _C), jnp.float32),
        ],
    )(adj, x, w1, w2)

    return (lsm.T[None], z.T[None])


# speedup vs baseline: 1.0506x; 1.0506x over previous
import jax
import jax.numpy as jnp
from jax.experimental import pallas as pl
from jax.experimental.pallas import tpu as pltpu

_N = 10000
_F = 128
_H = 32
_C = 8
_TM = 400
_NB = _N // _TM


def _fused_kernel(adj_ref, x_ref, w1_ref, w2_ref, lsm_ref, z_ref,
                  s1_ref, h2_ref, cache_ref):
    p = pl.program_id(0)
    i = pl.program_id(1)

    @pl.when((p == 0) & (i == 0))
    def _():
        s1_ref[...] = jnp.dot(x_ref[...], w1_ref[...],
                              preferred_element_type=jnp.float32)

    @pl.when(p == 0)
    def _():
        h = jnp.maximum(
            jnp.dot(adj_ref[...], s1_ref[...],
                    preferred_element_type=jnp.float32), 0.0)
        h2_ref[pl.ds(i * _TM, _TM), :] = jnp.dot(
            h, w2_ref[...], preferred_element_type=jnp.float32)

    @pl.when((p == 0) & (i == _NB - 2))
    def _():
        cache_ref[...] = adj_ref[...]

    def _layer2(src):
        z = jnp.dot(src, h2_ref[...], preferred_element_type=jnp.float32)
        z_ref[...] = z
        m = jnp.max(z, axis=1, keepdims=True)
        lse = jnp.log(jnp.sum(jnp.exp(z - m), axis=1, keepdims=True)) + m
        lsm_ref[...] = z - lse

    @pl.when((p == 1) & (i != _NB - 1))
    def _():
        _layer2(adj_ref[...])

    @pl.when((p == 1) & (i == _NB - 1))
    def _():
        _layer2(cache_ref[...])


def _adj_index(p, i):
    return (jnp.where(p == 0, i,
                      jnp.where(i == 0, _NB - 1,
                                jnp.where(i == _NB - 1, _NB - 3, i - 1))), 0)


def _out_index(p, i):
    return (jnp.where(p == 0, 0,
                      jnp.where(i == 0, _NB - 1,
                                jnp.where(i == _NB - 1, _NB - 2, i - 1))), 0)


def kernel(x, adj, W1, W2):
    w1 = W1.reshape(_F, _H)
    w2 = W2.reshape(_H, _C)

    lsm, z = pl.pallas_call(
        _fused_kernel,
        grid=(2, _NB),
        in_specs=[
            pl.BlockSpec((_TM, _N), _adj_index),
            pl.BlockSpec((_N, _F), lambda p, i: (0, 0)),
            pl.BlockSpec((_F, _H), lambda p, i: (0, 0)),
            pl.BlockSpec((_H, _C), lambda p, i: (0, 0)),
        ],
        out_specs=[
            pl.BlockSpec((_TM, _C), _out_index),
            pl.BlockSpec((_TM, _C), _out_index),
        ],
        out_shape=[
            jax.ShapeDtypeStruct((_N, _C), jnp.float32),
            jax.ShapeDtypeStruct((_N, _C), jnp.float32),
        ],
        compiler_params=pltpu.CompilerParams(
            vmem_limit_bytes=128 * 1024 * 1024),
        scratch_shapes=[
            pltpu.VMEM((_N, _H), jnp.float32),
            pltpu.VMEM((_N, _C), jnp.float32),
            pltpu.VMEM((_TM, _N), jnp.float32),
        ],
    )(adj, x, w1, w2)

    return (lsm.T[None], z.T[None])
